# Initial kernel scaffold; baseline (speedup 1.0000x reference)
import jax
import jax.numpy as jnp
from jax.experimental import pallas as pl


def kernel(x, system_id, W, b):
    raise NotImplementedError("write your pallas kernel here")


# speedup vs baseline: 1.9103x; 1.9103x over previous
import functools

import jax
import jax.numpy as jnp
from jax.experimental import pallas as pl

_B, _T, _D, _E, _C = 4096, 16, 1024, 8, 256
_BS = 256


def _fused_body(sid_ref, x_ref, w_ref, b_ref, o_ref):
    xp = jnp.mean(x_ref[...], axis=1)
    sid = sid_ref[0, 0, :]
    acc = jnp.zeros((_BS, _C), jnp.float32)
    for e in range(_E):
        mask = (sid == e).astype(jnp.float32)[:, None]
        y = jax.lax.dot_general(
            xp, w_ref[e],
            dimension_numbers=(((1,), (1,)), ((), ())),
            preferred_element_type=jnp.float32,
        )
        acc = acc + mask * (y + b_ref[e][None, :])
    o_ref[...] = acc


@jax.jit
def kernel(x, system_id, W, b):
    nb = _B // _BS
    sid3 = system_id.astype(jnp.int32).reshape(nb, 1, _BS)
    out = pl.pallas_call(
        _fused_body,
        grid=(nb,),
        in_specs=[
            pl.BlockSpec((1, 1, _BS), lambda i: (i, 0, 0)),
            pl.BlockSpec((_BS, _T, _D), lambda i: (i, 0, 0)),
            pl.BlockSpec((_E, _C, _D), lambda i: (0, 0, 0)),
            pl.BlockSpec((_E, _C), lambda i: (0, 0)),
        ],
        out_specs=pl.BlockSpec((_BS, _C), lambda i: (i, 0)),
        out_shape=jax.ShapeDtypeStruct((_B, _C), jnp.float32),
    )(sid3, x, W, b)
    return out
